# baseline (speedup 1.0000x reference)
import functools

import numpy as np
import jax
import jax.numpy as jnp
from jax import lax
from jax.experimental import pallas as pl
from jax.experimental.pallas import tpu as pltpu

_IN_DIM = 512
_OUT_DIM = 128
_UP_K = 10
_UP_L = 30
_DOWN_K = 60
_DOWN_L = 10
_BATCH = 256
_LL = [_IN_DIM] + [_DOWN_K] * _DOWN_L + [_UP_K] * _UP_L + [_OUT_DIM]
_LS = [int(s) for s in np.cumsum(_LL)]
_NLAYERS = len(_LL) - 1
_CC_TOTAL = 2 * _LS[_NLAYERS - 1]

_PREC = lax.Precision.DEFAULT


def _dot(a, b):
    return jnp.dot(a, b, precision=_PREC, preferred_element_type=jnp.float32)


def _threefry2x32(k1, k2, x0, x1):
    k1 = np.uint32(k1)
    k2 = np.uint32(k2)
    x0 = x0.astype(np.uint32).copy()
    x1 = x1.astype(np.uint32).copy()
    ks = [k1, k2, np.uint32(k1 ^ k2 ^ np.uint32(0x1BD11BDA))]
    rots = [(13, 15, 26, 6), (17, 29, 16, 24)]

    def rl(v, d):
        return (v << np.uint32(d)) | (v >> np.uint32(32 - d))

    with np.errstate(over="ignore"):
        x0 = x0 + ks[0]
        x1 = x1 + ks[1]
        for rnd in range(5):
            for r in rots[rnd % 2]:
                x0 = x0 + x1
                x1 = rl(x1, r)
                x1 = x0 ^ x1
            x0 = x0 + ks[(rnd + 1) % 3]
            x1 = x1 + ks[(rnd + 2) % 3] + np.uint32(rnd + 1)
    return x0, x1


def _np_fold_in(key, data):
    a, b = _threefry2x32(key[0], key[1], np.array([0], np.uint32),
                         np.array([data], np.uint32))
    return np.array([a[0], b[0]], np.uint32)


def _np_uniform(key, shape, minval, maxval):
    size = int(np.prod(shape))
    if size == 0:
        return np.zeros(shape, np.float32)
    b1, b2 = _threefry2x32(key[0], key[1], np.zeros(size, np.uint32),
                           np.arange(size, dtype=np.uint32))
    bits = b1 ^ b2
    fl = ((bits >> np.uint32(9)) | np.uint32(0x3F800000)).view(np.float32)
    mn = np.float32(minval)
    span = np.float32(maxval) - mn
    u = ((fl.astype(np.float64) - 1.0) * np.float64(span) + np.float64(mn)).astype(np.float32)
    return np.maximum(mn, u).reshape(shape)


@functools.lru_cache(maxsize=1)
def _static_constants():
    g_list, gwl_list = [], []
    gkey = np.array([0, 42], np.uint32)

    def gumbel(cnt, shape):
        u = _np_uniform(_np_fold_in(gkey, cnt), shape, 1e-6, 1.0 - 1e-6)
        return -np.log(-np.log(u.astype(np.float64))).astype(np.float32)

    cnt = 0
    for i in range(_NLAYERS):
        n_out = _LL[i + 1]
        cum = _LS[i]
        down_end = min(cum, _LS[_DOWN_L])
        n_down = (down_end - _IN_DIM) // _DOWN_K
        n_up = max(0, cum - _LS[_DOWN_L]) // _UP_K
        pieces = [gumbel(cnt, (2, n_out, _IN_DIM * 2))]
        cnt += 1
        gd = gumbel(cnt, (2, n_out, n_down, _DOWN_K * 2))
        cnt += 1
        if n_down:
            pieces.append(gd.reshape(2, n_out, -1))
        gu = gumbel(cnt, (2, n_out, n_up, _UP_K * 2))
        cnt += 1
        if n_up:
            pieces.append(gu.reshape(2, n_out, -1))
        g = np.concatenate(pieces, axis=-1).reshape(2 * n_out, 2 * cum)
        gw = gumbel(cnt, (2, n_out, i + 1)).reshape(2 * n_out, i + 1)
        cnt += 1
        if i == _NLAYERS - 1:
            g = g[:n_out]
            gw = gw[:n_out]
        g_list.append(g)
        gwl_list.append(gw)

    bounds = [0, 2 * _IN_DIM]
    for _ in range(_DOWN_L):
        bounds.append(bounds[-1] + 2 * _DOWN_K)
    for _ in range(_UP_L):
        bounds.append(bounds[-1] + 2 * _UP_K)
    seg = np.zeros((_CC_TOTAL, _NLAYERS), np.float32)
    for gidx in range(_NLAYERS):
        seg[bounds[gidx]:bounds[gidx + 1], gidx] = 1.0
    segT = seg.T.copy()

    def interleave_mat(w):
        t = np.zeros((2 * w, w), np.float32)
        for j in range(w):
            t[2 * j, j] = -1.0
            t[2 * j + 1, j] = 1.0
        return t

    return g_list, gwl_list, seg, segT, interleave_mat(_DOWN_K), interleave_mat(_UP_K)


_static_constants()


_GROUPS = [list(range(0, 5)), list(range(5, 10)), list(range(10, 41))]
_NEG = -1e30


def _group_meta(group):
    last_group = group[-1] == _NLAYERS - 1
    rows = [(_LL[i + 1] if i == _NLAYERS - 1 else 2 * _LL[i + 1]) for i in group]
    roff = [0]
    for r in rows:
        roff.append(roff[-1] + r)
    cc_first = 2 * _LS[group[0]]
    cc_g = 2 * _LS[group[-1]]
    ng_g = group[-1] + 1
    return last_group, rows, roff, cc_first, cc_g, ng_g


def _group_body(group, refs):
    last_group, rows, roff, cc_first, cc_g, ng_g = _group_meta(group)
    rt = roff[-1]
    it = iter(refs)
    xeT_in = next(it)
    ab_ref = next(it)
    wl_ref = next(it)
    g_ref = next(it)
    gwl_ref = next(it)
    seg_ref = next(it)
    segT_ref = next(it)
    t_refs = {_DOWN_K: next(it), _UP_K: next(it)}
    out_ref = next(it)
    eg = next(it)
    xeT = next(it) if last_group else out_ref

    xeT[...] = xeT_in[...]

    for c0 in range(0, cc_g, 512):
        c1 = min(c0 + 512, cc_g)
        eg[0:rt, c0:c1] = jnp.exp(jnp.abs(ab_ref[0:rt, c0:c1]) + g_ref[0:rt, c0:c1])
    S = _dot(eg[0:rt, 0:cc_g], seg_ref[...])
    ew = jnp.exp(wl_ref[...] + gwl_ref[...])
    wl_sm = ew / jnp.sum(ew, axis=1, keepdims=True)
    R = wl_sm / jnp.maximum(S, 1e-30)
    for c0 in range(0, cc_g, 512):
        c1 = min(c0 + 512, cc_g)
        eg[0:rt, c0:c1] = eg[0:rt, c0:c1] * _dot(R, segT_ref[0:ng_g, c0:c1])

    z_base = _dot(eg[0:rt, 0:cc_first], xeT[0:cc_first, :])

    for j, i in enumerate(group):
        n_out = _LL[i + 1]
        cc = 2 * _LS[i]
        r0, r1 = roff[j], roff[j + 1]
        z = z_base[r0:r1, :]
        if j > 0:
            z = z + _dot(eg[r0:r1, cc_first:cc_g], xeT[cc_first:cc_g, :])
        if i == _NLAYERS - 1:
            out_ref[...] = z
        else:
            p = z[0:n_out, :] * z[n_out:2 * n_out, :]
            row_par = lax.broadcasted_iota(jnp.int32, (2 * n_out, _BATCH), 0) % 2
            cmask = (row_par == 0).astype(jnp.float32)
            xeT[cc:cc + 2 * n_out, :] = cmask + _dot(t_refs[n_out][...], p)


@functools.lru_cache(maxsize=1)
def _packed_constants():
    g_list, gwl_list, seg, segT, t_down, t_up = _static_constants()
    out = []
    for group in _GROUPS:
        last_group, rows, roff, cc_first, cc_g, ng_g = _group_meta(group)
        rt = roff[-1]
        gp = np.full((rt, cc_g), _NEG, np.float32)
        gwp = np.full((rt, ng_g), _NEG, np.float32)
        for j, i in enumerate(group):
            gp[roff[j]:roff[j + 1], 0:2 * _LS[i]] = g_list[i]
            gwp[roff[j]:roff[j + 1], 0:i + 1] = gwl_list[i]
        out.append((gp, gwp, seg[:cc_g, :ng_g].copy(), segT[:ng_g, :cc_g].copy()))
    return out, t_down, t_up


def _run_pallas(xe0, abr, wlr, interpret=False):
    packs, t_down, t_up = _packed_constants()
    t_down = jnp.asarray(t_down)
    t_up = jnp.asarray(t_up)

    xeT = jnp.concatenate(
        [xe0, jnp.zeros((_CC_TOTAL - 2 * _IN_DIM, _BATCH), jnp.float32)], axis=0
    )
    for gi, group in enumerate(_GROUPS):
        last_group, rows, roff, cc_first, cc_g, ng_g = _group_meta(group)
        rt = roff[-1]
        gp, gwp, seg_g, segT_g = packs[gi]
        ab_pack = jnp.concatenate(
            [jnp.pad(abr[i], ((0, 0), (0, cc_g - 2 * _LS[i]))) for i in group], axis=0
        )
        wl_pack = jnp.concatenate(
            [jnp.pad(wlr[i], ((0, 0), (0, ng_g - (i + 1)))) for i in group], axis=0
        )
        inputs = [xeT, ab_pack, wl_pack, jnp.asarray(gp), jnp.asarray(gwp),
                  jnp.asarray(seg_g), jnp.asarray(segT_g), t_down, t_up]
        scratch = [pltpu.VMEM((rt, cc_g), jnp.float32)]
        if last_group:
            out_shape = jax.ShapeDtypeStruct((_OUT_DIM, _BATCH), jnp.float32)
            scratch.append(pltpu.VMEM((_CC_TOTAL, _BATCH), jnp.float32))
        else:
            out_shape = jax.ShapeDtypeStruct((_CC_TOTAL, _BATCH), jnp.float32)

        def body(*refs, _group=tuple(group)):
            _group_body(list(_group), refs)

        xeT = pl.pallas_call(
            body,
            out_shape=out_shape,
            scratch_shapes=scratch,
            interpret=interpret,
        )(*inputs)
    return xeT


def kernel(x, layers, which_layers):
    xT = jnp.transpose(x)
    xe0 = jnp.stack([xT, 1.0 - xT], axis=1).reshape(2 * _IN_DIM, _BATCH)
    abr, wlr = [], []
    for i in range(_NLAYERS):
        n_out = _LL[i + 1]
        a = layers[i].reshape(2 * n_out, 2 * _LS[i])
        w = which_layers[i].reshape(2 * n_out, i + 1)
        if i == _NLAYERS - 1:
            a = a[0:n_out]
            w = w[0:n_out]
        abr.append(a)
        wlr.append(w)
    outT = _run_pallas(xe0, abr, wlr)
    return jnp.transpose(outT)


# speedup vs baseline: 1.3096x; 1.3096x over previous
import functools

import numpy as np
import jax
import jax.numpy as jnp
from jax import lax
from jax.experimental import pallas as pl
from jax.experimental.pallas import tpu as pltpu

_IN_DIM = 512
_OUT_DIM = 128
_UP_K = 10
_UP_L = 30
_DOWN_K = 60
_DOWN_L = 10
_BATCH = 256
_LL = [_IN_DIM] + [_DOWN_K] * _DOWN_L + [_UP_K] * _UP_L + [_OUT_DIM]
_LS = [int(s) for s in np.cumsum(_LL)]
_NLAYERS = len(_LL) - 1
_CC_TOTAL = 2 * _LS[_NLAYERS - 1]

_PREC = lax.Precision.DEFAULT


def _dot(a, b):
    return jnp.dot(a, b, precision=_PREC, preferred_element_type=jnp.float32)


def _threefry2x32(k1, k2, x0, x1):
    k1 = np.uint32(k1)
    k2 = np.uint32(k2)
    x0 = x0.astype(np.uint32).copy()
    x1 = x1.astype(np.uint32).copy()
    ks = [k1, k2, np.uint32(k1 ^ k2 ^ np.uint32(0x1BD11BDA))]
    rots = [(13, 15, 26, 6), (17, 29, 16, 24)]

    def rl(v, d):
        return (v << np.uint32(d)) | (v >> np.uint32(32 - d))

    with np.errstate(over="ignore"):
        x0 = x0 + ks[0]
        x1 = x1 + ks[1]
        for rnd in range(5):
            for r in rots[rnd % 2]:
                x0 = x0 + x1
                x1 = rl(x1, r)
                x1 = x0 ^ x1
            x0 = x0 + ks[(rnd + 1) % 3]
            x1 = x1 + ks[(rnd + 2) % 3] + np.uint32(rnd + 1)
    return x0, x1


def _np_fold_in(key, data):
    a, b = _threefry2x32(key[0], key[1], np.array([0], np.uint32),
                         np.array([data], np.uint32))
    return np.array([a[0], b[0]], np.uint32)


def _np_uniform(key, shape, minval, maxval):
    size = int(np.prod(shape))
    if size == 0:
        return np.zeros(shape, np.float32)
    b1, b2 = _threefry2x32(key[0], key[1], np.zeros(size, np.uint32),
                           np.arange(size, dtype=np.uint32))
    bits = b1 ^ b2
    fl = ((bits >> np.uint32(9)) | np.uint32(0x3F800000)).view(np.float32)
    mn = np.float32(minval)
    span = np.float32(maxval) - mn
    u = ((fl.astype(np.float64) - 1.0) * np.float64(span) + np.float64(mn)).astype(np.float32)
    return np.maximum(mn, u).reshape(shape)


@functools.lru_cache(maxsize=1)
def _static_constants():
    g_list, gwl_list = [], []
    gkey = np.array([0, 42], np.uint32)

    def gumbel(cnt, shape):
        u = _np_uniform(_np_fold_in(gkey, cnt), shape, 1e-6, 1.0 - 1e-6)
        return -np.log(-np.log(u.astype(np.float64))).astype(np.float32)

    cnt = 0
    for i in range(_NLAYERS):
        n_out = _LL[i + 1]
        cum = _LS[i]
        down_end = min(cum, _LS[_DOWN_L])
        n_down = (down_end - _IN_DIM) // _DOWN_K
        n_up = max(0, cum - _LS[_DOWN_L]) // _UP_K
        pieces = [gumbel(cnt, (2, n_out, _IN_DIM * 2))]
        cnt += 1
        gd = gumbel(cnt, (2, n_out, n_down, _DOWN_K * 2))
        cnt += 1
        if n_down:
            pieces.append(gd.reshape(2, n_out, -1))
        gu = gumbel(cnt, (2, n_out, n_up, _UP_K * 2))
        cnt += 1
        if n_up:
            pieces.append(gu.reshape(2, n_out, -1))
        g = np.concatenate(pieces, axis=-1).reshape(2 * n_out, 2 * cum)
        gw = gumbel(cnt, (2, n_out, i + 1)).reshape(2 * n_out, i + 1)
        cnt += 1
        if i == _NLAYERS - 1:
            g = g[:n_out]
            gw = gw[:n_out]
        g_list.append(g)
        gwl_list.append(gw)

    bounds = [0, 2 * _IN_DIM]
    for _ in range(_DOWN_L):
        bounds.append(bounds[-1] + 2 * _DOWN_K)
    for _ in range(_UP_L):
        bounds.append(bounds[-1] + 2 * _UP_K)
    seg = np.zeros((_CC_TOTAL, _NLAYERS), np.float32)
    for gidx in range(_NLAYERS):
        seg[bounds[gidx]:bounds[gidx + 1], gidx] = 1.0
    segT = seg.T.copy()

    def interleave_mat(w):
        t = np.zeros((2 * w, w), np.float32)
        for j in range(w):
            t[2 * j, j] = -1.0
            t[2 * j + 1, j] = 1.0
        return t

    return g_list, gwl_list, seg, segT, interleave_mat(_DOWN_K), interleave_mat(_UP_K)


_static_constants()


_GROUPS = [list(range(0, 5)), list(range(5, 10)), list(range(10, 25)),
           list(range(25, 40)), [40]]


def _group_body(group, refs):
    k = len(group)
    it = iter(refs)
    xeT_in = next(it)
    abr_refs = [next(it) for _ in range(k)]
    wlr_refs = [next(it) for _ in range(k)]
    g_refs = [next(it) for _ in range(k)]
    gwl_refs = [next(it) for _ in range(k)]
    seg_ref = next(it)
    segT_ref = next(it)
    last_group = group[-1] == _NLAYERS - 1
    t_refs = {} if last_group else {_DOWN_K: next(it), _UP_K: next(it)}
    out_ref = next(it)
    xeT = next(it) if last_group else out_ref

    xeT[...] = xeT_in[...]
    for j, i in enumerate(group):
        n_out = _LL[i + 1]
        cc = 2 * _LS[i]
        ng = i + 1
        last = i == _NLAYERS - 1

        E = jnp.exp(jnp.abs(abr_refs[j][...]) + g_refs[j][...])
        ew = jnp.exp(wlr_refs[j][...] + gwl_refs[j][...])
        wl_sm = ew / jnp.sum(ew, axis=1, keepdims=True)
        S = _dot(E, seg_ref[0:cc, 0:ng])
        R = wl_sm / jnp.maximum(S, 1e-30)
        Rfull = _dot(R, segT_ref[0:ng, 0:cc])
        W = E * Rfull
        z = _dot(W, xeT[0:cc, :])
        if last:
            out_ref[...] = z
        else:
            p = z[0:n_out, :] * z[n_out:2 * n_out, :]
            row_par = lax.broadcasted_iota(jnp.int32, (2 * n_out, _BATCH), 0) % 2
            cmask = (row_par == 0).astype(jnp.float32)
            xeT[cc:cc + 2 * n_out, :] = cmask + _dot(t_refs[n_out][...], p)


def _run_pallas(xe0, abr, wlr, interpret=False):
    g_list, gwl_list, seg, segT, t_down, t_up = _static_constants()
    seg = jnp.asarray(seg)
    segT = jnp.asarray(segT)
    t_down = jnp.asarray(t_down)
    t_up = jnp.asarray(t_up)

    xeT = jnp.concatenate(
        [xe0, jnp.zeros((_CC_TOTAL - 2 * _IN_DIM, _BATCH), jnp.float32)], axis=0
    )
    for group in _GROUPS:
        last_group = group[-1] == _NLAYERS - 1
        inputs = (
            [xeT]
            + [abr[i] for i in group]
            + [wlr[i] for i in group]
            + [jnp.asarray(g_list[i]) for i in group]
            + [jnp.asarray(gwl_list[i]) for i in group]
            + [seg, segT]
            + ([] if last_group else [t_down, t_up])
        )
        if last_group:
            out_shape = jax.ShapeDtypeStruct((_OUT_DIM, _BATCH), jnp.float32)
            scratch = [pltpu.VMEM((_CC_TOTAL, _BATCH), jnp.float32)]
        else:
            out_shape = jax.ShapeDtypeStruct((_CC_TOTAL, _BATCH), jnp.float32)
            scratch = []

        def body(*refs, _group=tuple(group)):
            _group_body(list(_group), refs)

        xeT = pl.pallas_call(
            body,
            out_shape=out_shape,
            scratch_shapes=scratch,
            interpret=interpret,
        )(*inputs)
    return xeT


def kernel(x, layers, which_layers):
    xT = jnp.transpose(x)
    xe0 = jnp.stack([xT, 1.0 - xT], axis=1).reshape(2 * _IN_DIM, _BATCH)
    abr, wlr = [], []
    for i in range(_NLAYERS):
        n_out = _LL[i + 1]
        a = layers[i].reshape(2 * n_out, 2 * _LS[i])
        w = which_layers[i].reshape(2 * n_out, i + 1)
        if i == _NLAYERS - 1:
            a = a[0:n_out]
            w = w[0:n_out]
        abr.append(a)
        wlr.append(w)
    outT = _run_pallas(xe0, abr, wlr)
    return jnp.transpose(outT)
